# baseline (speedup 1.0000x reference)
import functools

import jax
import jax.numpy as jnp
from jax import lax
from jax.experimental import pallas as pl
from jax.experimental.pallas import tpu as pltpu
from jax.experimental.pallas import tpu_sc as plsc

BATCH = 4096
SEQ = 200
D = 64
VOCAB = 1000000
NC, NS = 2, 16
NW = NC * NS
CPW = BATCH // NW
NBUF = 4
NJ = (VOCAB + 127) // 128
NJPW = (NJ + NW - 1) // NW

_mesh = plsc.VectorSubcoreMesh(core_axis_name="c", subcore_axis_name="s")
_params = pltpu.CompilerParams(needs_layout_passes=False)


@functools.partial(
    pl.kernel,
    mesh=_mesh,
    out_type=jax.ShapeDtypeStruct((VOCAB // 2, 2 * D), jnp.float32),
    compiler_params=_params,
    scratch_types=[
        pltpu.VMEM((NBUF, D, 128), jnp.float32),
        pltpu.VMEM((NBUF, D, 128), jnp.float32),
        pltpu.SemaphoreType.DMA((NBUF,)),
        pltpu.SemaphoreType.DMA((NBUF,)),
    ],
)
def _pack(tt_hbm, tail_hbm, p_hbm, inb, outb, isem, osem):
    wid = lax.axis_index("s") * NC + lax.axis_index("c")

    rowv = [lax.iota(jnp.int32, 16) + (16 * (q % 4)) for q in range(8)]

    def din(j, b):
        pltpu.async_copy(tt_hbm.at[:, pl.ds(j * 128, 128)], inb.at[b],
                         isem.at[b])

    def din_tail(b):
        pltpu.async_copy(tail_hbm, inb.at[b], isem.at[b])

    def din_wait(b):
        pltpu.make_async_copy(tt_hbm.at[:, pl.ds(0, 128)], inb.at[b],
                              isem.at[b]).wait()

    def transpose(b, nrows, coff=0):
        for c2 in range(nrows):
            for q in range(8):
                col = jnp.full((16,), coff + 2 * c2 + q // 4, jnp.int32)
                val = plsc.load_gather(inb.at[b], [rowv[q], col])
                outb.at[b][c2, pl.ds(16 * q, 16)] = val

    def dout(j, b):
        pltpu.async_copy(outb.at[b], p_hbm.at[pl.ds(j * 64, 64), :],
                         osem.at[b])

    def dout_tail(b):
        pltpu.async_copy(outb.at[b, pl.ds(0, 32)],
                         p_hbm.at[pl.ds((NJ - 1) * 64, 32), :], osem.at[b])

    def dout_wait(b):
        pltpu.make_async_copy(outb.at[b], p_hbm.at[pl.ds(0, 64), :],
                              osem.at[b]).wait()

    def dout_wait_tail(b):
        pltpu.make_async_copy(outb.at[b, pl.ds(0, 32)],
                              p_hbm.at[pl.ds(0, 32), :], osem.at[b]).wait()

    for b in range(NBUF):
        din(wid + NW * b, b)

    def body(jj, carry):
        j = wid + NW * jj
        slot = lax.rem(jj, NBUF)

        @pl.when(j < NJ - 1)
        def _full():
            din_wait(slot)

            @pl.when(jj >= NBUF)
            def _():
                dout_wait(slot)

            transpose(slot, 64)
            dout(j, slot)

        @pl.when(j == NJ - 1)
        def _tail():
            din_wait(slot)

            @pl.when(jj >= NBUF)
            def _():
                dout_wait(slot)

            transpose(slot, 32, coff=64)
            dout_tail(slot)

        jn = j + NW * NBUF

        @pl.when(jn < NJ - 1)
        def _next():
            din(jn, slot)

        @pl.when(jn == NJ - 1)
        def _next_tail():
            din_tail(slot)

        return carry

    lax.fori_loop(0, NJPW, body, 0)

    for k in range(NBUF - 1, 0, -1):
        jj = NJPW - 1 - k
        slot_k = (NJPW - 1 - k) % NBUF

        @pl.when(wid + NW * jj == NJ - 1)
        def _():
            dout_wait_tail(slot_k)

        @pl.when(wid + NW * jj < NJ - 1)
        def _():
            dout_wait(slot_k)

    last_slot = (NJPW - 1) % NBUF

    @pl.when(wid + NW * (NJPW - 1) == NJ - 1)
    def _():
        dout_wait_tail(last_slot)

    @pl.when(wid + NW * (NJPW - 1) < NJ - 1)
    def _():
        dout_wait(last_slot)


@functools.partial(
    pl.kernel,
    mesh=_mesh,
    out_type=jax.ShapeDtypeStruct((SEQ, D, BATCH), jnp.float32),
    compiler_params=_params,
    scratch_types=[
        pltpu.VMEM((SEQ, CPW), jnp.int32),
        pltpu.VMEM((NBUF, CPW), jnp.int32),
        pltpu.VMEM((NBUF, CPW, 128), jnp.float32),
        pltpu.VMEM((NBUF, D, CPW), jnp.float32),
        pltpu.SemaphoreType.DMA((NBUF,)),
        pltpu.SemaphoreType.DMA((NBUF,)),
    ],
)
def _emb_lookup(ids_hbm, table_hbm, out_hbm, ids_v, idx_v, gbuf, obuf,
                gsem, ssem):
    wid = lax.axis_index("s") * NC + lax.axis_index("c")
    base = wid * CPW
    pltpu.sync_copy(ids_hbm.at[:, pl.ds(base, CPW)], ids_v)

    rowidx = [lax.iota(jnp.int32, 16) + (16 * g) for g in range(8)]

    def prep(s, b):
        for g in range(8):
            vec = ids_v.at[s][pl.ds(16 * g, 16)]
            idx_v.at[b][pl.ds(16 * g, 16)] = lax.shift_right_logical(vec, 1)

    def gather(b):
        pltpu.async_copy(table_hbm.at[idx_v.at[b]], gbuf.at[b], gsem.at[b])

    def gather_wait(b):
        pltpu.make_async_copy(table_hbm.at[idx_v.at[b]], gbuf.at[b],
                              gsem.at[b]).wait()

    def store(s, b):
        pltpu.async_copy(obuf.at[b], out_hbm.at[s, :, pl.ds(base, CPW)],
                         ssem.at[b])

    def store_wait(b):
        pltpu.make_async_copy(obuf.at[b], out_hbm.at[0, :, pl.ds(base, CPW)],
                              ssem.at[b]).wait()

    def transpose(s, b):
        gb = gbuf.at[b]
        ob = obuf.at[b]
        halves = [
            (ids_v.at[s][pl.ds(16 * g, 16)] & 1) * 64 for g in range(8)
        ]
        for d in range(D):
            for g in range(8):
                val = plsc.load_gather(gb, [rowidx[g], halves[g] + d])
                ob.at[d][pl.ds(16 * g, 16)] = val

    for b in range(NBUF):
        prep(b, b)
        gather(b)

    def body(s, carry):
        slot = lax.rem(s, NBUF)
        gather_wait(slot)

        @pl.when(s >= NBUF)
        def _():
            store_wait(slot)

        transpose(s, slot)
        store(s, slot)

        @pl.when(s < SEQ - NBUF)
        def _():
            prep(s + NBUF, slot)
            gather(slot)

        return carry

    lax.fori_loop(0, SEQ, body, 0)

    for b in range(NBUF):
        store_wait(b)


def kernel(token_ids, embed_mat):
    ids_t = token_ids.T
    emb_t = embed_mat.T
    tail = lax.slice(emb_t, (0, VOCAB - 128), (D, VOCAB))
    packed = _pack(emb_t, tail)
    out_t = _emb_lookup(ids_t, packed)
    return jnp.transpose(out_t, (2, 0, 1))


# speedup vs baseline: 2.7268x; 2.7268x over previous
import functools

import jax
import jax.numpy as jnp
from jax import lax
from jax.experimental import pallas as pl
from jax.experimental.pallas import tpu as pltpu
from jax.experimental.pallas import tpu_sc as plsc

BATCH = 4096
SEQ = 200
D = 64
VOCAB = 1000000
NC, NS = 2, 16
NW = NC * NS
RPW = BATCH // NW
NBUF = 4
NROUNDS = SEQ // NBUF

_mesh = plsc.VectorSubcoreMesh(core_axis_name="c", subcore_axis_name="s")


@functools.partial(
    pl.kernel,
    mesh=_mesh,
    out_type=jax.ShapeDtypeStruct((BATCH, SEQ, 2 * D), jnp.float32),
    compiler_params=pltpu.CompilerParams(needs_layout_passes=False),
    scratch_types=[
        pltpu.VMEM((SEQ, RPW), jnp.int32),
        pltpu.VMEM((NBUF, RPW, 2 * D), jnp.float32),
        pltpu.SemaphoreType.DMA((NBUF,)),
        pltpu.SemaphoreType.DMA((NBUF,)),
    ],
)
def _emb_lookup(ids_hbm, table_hbm, out_hbm, ids_v, gbuf, gsem, ssem):
    wid = lax.axis_index("s") * NC + lax.axis_index("c")
    base = wid * RPW
    pltpu.sync_copy(ids_hbm.at[:, pl.ds(base, RPW)], ids_v)

    def gather(s, b):
        pltpu.async_copy(table_hbm.at[ids_v.at[s]], gbuf.at[b], gsem.at[b])

    def gather_wait(b):
        pltpu.make_async_copy(table_hbm.at[ids_v.at[0]], gbuf.at[b],
                              gsem.at[b]).wait()

    def store(s, b):
        pltpu.async_copy(gbuf.at[b], out_hbm.at[pl.ds(base, RPW), s],
                         ssem.at[b])

    def store_wait(b):
        pltpu.make_async_copy(gbuf.at[b], out_hbm.at[pl.ds(base, RPW), 0],
                              ssem.at[b]).wait()

    for b in range(NBUF):
        gather(b, b)

    def body(r, carry):
        s0 = r * NBUF
        for b in range(NBUF):
            gather_wait(b)
            store(s0 + b, b)
        for b in range(NBUF):
            store_wait(b)
            gather(s0 + NBUF + b, b)
        return carry

    lax.fori_loop(0, NROUNDS - 1, body, 0)

    s0 = (NROUNDS - 1) * NBUF
    for b in range(NBUF):
        gather_wait(b)
        store(s0 + b, b)
    for b in range(NBUF):
        store_wait(b)


def kernel(token_ids, embed_mat):
    padded = jnp.pad(embed_mat, ((0, 0), (0, D)))
    wide = _emb_lookup(token_ids.T, padded)
    return wide[:, :, :D]


# speedup vs baseline: 2.7294x; 1.0010x over previous
import functools

import jax
import jax.numpy as jnp
from jax import lax
from jax.experimental import pallas as pl
from jax.experimental.pallas import tpu as pltpu
from jax.experimental.pallas import tpu_sc as plsc

BATCH = 4096
SEQ = 200
D = 64
VOCAB = 1000000
NC, NS = 2, 16
NW = NC * NS
RPW = BATCH // NW
NBUF = 5
NROUNDS = SEQ // NBUF

_mesh = plsc.VectorSubcoreMesh(core_axis_name="c", subcore_axis_name="s")


@functools.partial(
    pl.kernel,
    mesh=_mesh,
    out_type=jax.ShapeDtypeStruct((BATCH, SEQ, 2 * D), jnp.float32),
    compiler_params=pltpu.CompilerParams(needs_layout_passes=False),
    scratch_types=[
        pltpu.VMEM((SEQ, RPW), jnp.int32),
        pltpu.VMEM((NBUF, RPW, 2 * D), jnp.float32),
        pltpu.SemaphoreType.DMA((NBUF,)),
        pltpu.SemaphoreType.DMA((NBUF,)),
    ],
)
def _emb_lookup(ids_hbm, table_hbm, out_hbm, ids_v, gbuf, gsem, ssem):
    wid = lax.axis_index("s") * NC + lax.axis_index("c")
    base = wid * RPW
    pltpu.sync_copy(ids_hbm.at[:, pl.ds(base, RPW)], ids_v)

    def gather(s, b):
        pltpu.async_copy(table_hbm.at[ids_v.at[s]], gbuf.at[b], gsem.at[b])

    def gather_wait(b):
        pltpu.make_async_copy(table_hbm.at[ids_v.at[0]], gbuf.at[b],
                              gsem.at[b]).wait()

    def store(s, b):
        pltpu.async_copy(gbuf.at[b], out_hbm.at[pl.ds(base, RPW), s],
                         ssem.at[b])

    def store_wait(b):
        pltpu.make_async_copy(gbuf.at[b], out_hbm.at[pl.ds(base, RPW), 0],
                              ssem.at[b]).wait()

    for b in range(NBUF):
        gather(b, b)

    def body(r, carry):
        s0 = r * NBUF
        for b in range(NBUF):
            gather_wait(b)
            store(s0 + b, b)
        for b in range(NBUF):
            store_wait(b)
            gather(s0 + NBUF + b, b)
        return carry

    lax.fori_loop(0, NROUNDS - 1, body, 0)

    s0 = (NROUNDS - 1) * NBUF
    for b in range(NBUF):
        gather_wait(b)
        store(s0 + b, b)
    for b in range(NBUF):
        store_wait(b)


def kernel(token_ids, embed_mat):
    padded = jnp.pad(embed_mat, ((0, 0), (0, D)))
    wide = _emb_lookup(token_ids.T, padded)
    return wide[:, :, :D]
